# baseline (device time: 72036 ns/iter reference)
import jax
import jax.numpy as jnp
from jax import lax
from jax.experimental import pallas as pl
from jax.experimental.pallas import tpu as pltpu

N_DEV = 4
DH = 64


def _ring_allreduce(partial):
    m, n = partial.shape

    def body(p_ref, out_ref, comm, send_sems, recv_sems):
        my = lax.axis_index("i")
        left = lax.rem(my + N_DEV - 1, N_DEV)
        right = lax.rem(my + 1, N_DEV)

        barrier_sem = pltpu.get_barrier_semaphore()
        for nbr in (left, right):
            pl.semaphore_signal(
                barrier_sem, inc=1,
                device_id=(nbr,), device_id_type=pl.DeviceIdType.MESH,
            )
        pl.semaphore_wait(barrier_sem, 2)

        out_ref[...] = p_ref[...]
        comm[0] = p_ref[...]

        for h in range(N_DEV - 1):
            rdma = pltpu.make_async_remote_copy(
                src_ref=comm.at[h],
                dst_ref=comm.at[h + 1],
                send_sem=send_sems.at[h],
                recv_sem=recv_sems.at[h],
                device_id=(right,),
                device_id_type=pl.DeviceIdType.MESH,
            )
            rdma.start()
            rdma.wait()
            out_ref[...] += comm[h + 1]

    return pl.pallas_call(
        body,
        out_shape=jax.ShapeDtypeStruct((m, n), jnp.float32),
        in_specs=[pl.BlockSpec(memory_space=pltpu.VMEM)],
        out_specs=pl.BlockSpec(memory_space=pltpu.VMEM),
        scratch_shapes=[
            pltpu.VMEM((N_DEV, m, n), jnp.float32),
            pltpu.SemaphoreType.DMA((N_DEV - 1,)),
            pltpu.SemaphoreType.DMA((N_DEV - 1,)),
        ],
        compiler_params=pltpu.CompilerParams(collective_id=0),
    )(partial)


def kernel(x, Wq, Wo, K_ext, V_ext):
    my = lax.axis_index("i")
    B, Sq, D = x.shape
    Hq = Wq.shape[1] // DH

    xb = x.astype(jnp.bfloat16)
    Q = (xb @ Wq.astype(jnp.bfloat16)).reshape(B, Sq, Hq, DH)
    K = lax.dynamic_slice_in_dim(K_ext, my * Hq, Hq, axis=2).astype(jnp.bfloat16)
    V = lax.dynamic_slice_in_dim(V_ext, my * Hq, Hq, axis=2).astype(jnp.bfloat16)

    s = jnp.einsum(
        "bihd,bjhd->bhij", Q, K, preferred_element_type=jnp.float32
    ) * 0.125
    p = jax.nn.softmax(s, axis=-1).astype(jnp.bfloat16)
    o = jnp.einsum(
        "bhij,bjhd->bihd", p, V, preferred_element_type=jnp.float32
    )
    o = o.astype(jnp.bfloat16).reshape(B, Sq, Hq * DH)
    partial = jnp.dot(
        o, Wo.astype(jnp.bfloat16), preferred_element_type=jnp.float32
    )

    out = _ring_allreduce(partial.reshape(B * Sq, D))
    return out.reshape(B, Sq, D)


# device time: 26861 ns/iter; 2.6818x vs baseline; 2.6818x over previous
import jax
import jax.numpy as jnp
from jax import lax
from jax.experimental import pallas as pl
from jax.experimental.pallas import tpu as pltpu

N_DEV = 4
DH = 64


def _butterfly_allreduce(partial):
    m, n = partial.shape
    h = m // 2

    def body(p_ref, out_ref, cA, cB, sA, rA, sB, rB):
        my = lax.axis_index("i")
        p1 = my ^ 1
        p2 = 3 - my

        barrier_sem = pltpu.get_barrier_semaphore()
        for nbr in (p1, p2):
            pl.semaphore_signal(
                barrier_sem, inc=1,
                device_id=(nbr,), device_id_type=pl.DeviceIdType.MESH,
            )
        pl.semaphore_wait(barrier_sem, 2)

        cA[0] = p_ref[pl.ds(0, h), :]
        cB[0] = p_ref[pl.ds(h, h), :]

        a1 = pltpu.make_async_remote_copy(
            src_ref=cA.at[0], dst_ref=cA.at[1],
            send_sem=sA.at[0], recv_sem=rA.at[0],
            device_id=(p1,), device_id_type=pl.DeviceIdType.MESH,
        )
        b1 = pltpu.make_async_remote_copy(
            src_ref=cB.at[0], dst_ref=cB.at[1],
            send_sem=sB.at[0], recv_sem=rB.at[0],
            device_id=(p2,), device_id_type=pl.DeviceIdType.MESH,
        )
        a1.start()
        b1.start()
        a1.wait()
        b1.wait()

        pairA = cA[0].astype(jnp.float32) + cA[1].astype(jnp.float32)
        pairB = cB[0].astype(jnp.float32) + cB[1].astype(jnp.float32)
        cA[2] = pairA.astype(jnp.bfloat16)
        cB[2] = pairB.astype(jnp.bfloat16)

        a2 = pltpu.make_async_remote_copy(
            src_ref=cA.at[2], dst_ref=cA.at[3],
            send_sem=sA.at[1], recv_sem=rA.at[1],
            device_id=(p2,), device_id_type=pl.DeviceIdType.MESH,
        )
        b2 = pltpu.make_async_remote_copy(
            src_ref=cB.at[2], dst_ref=cB.at[3],
            send_sem=sB.at[1], recv_sem=rB.at[1],
            device_id=(p1,), device_id_type=pl.DeviceIdType.MESH,
        )
        a2.start()
        b2.start()
        a2.wait()
        b2.wait()

        out_ref[pl.ds(0, h), :] = pairA + cA[3].astype(jnp.float32)
        out_ref[pl.ds(h, h), :] = pairB + cB[3].astype(jnp.float32)

    return pl.pallas_call(
        body,
        out_shape=jax.ShapeDtypeStruct((m, n), jnp.float32),
        in_specs=[pl.BlockSpec(memory_space=pltpu.VMEM)],
        out_specs=pl.BlockSpec(memory_space=pltpu.VMEM),
        scratch_shapes=[
            pltpu.VMEM((4, h, n), jnp.bfloat16),
            pltpu.VMEM((4, h, n), jnp.bfloat16),
            pltpu.SemaphoreType.DMA((2,)),
            pltpu.SemaphoreType.DMA((2,)),
            pltpu.SemaphoreType.DMA((2,)),
            pltpu.SemaphoreType.DMA((2,)),
        ],
        compiler_params=pltpu.CompilerParams(collective_id=0),
    )(partial)


def kernel(x, Wq, Wo, K_ext, V_ext):
    my = lax.axis_index("i")
    B, Sq, D = x.shape
    Hq = Wq.shape[1] // DH

    xb = x.astype(jnp.bfloat16)
    Q = (xb @ Wq.astype(jnp.bfloat16)).reshape(B, Sq, Hq, DH)
    K = lax.dynamic_slice_in_dim(K_ext, my * Hq, Hq, axis=2).astype(jnp.bfloat16)
    V = lax.dynamic_slice_in_dim(V_ext, my * Hq, Hq, axis=2).astype(jnp.bfloat16)

    s = jnp.einsum(
        "bihd,bjhd->bhij", Q, K, preferred_element_type=jnp.float32
    ) * 0.125
    p = jax.nn.softmax(s, axis=-1).astype(jnp.bfloat16)
    o = jnp.einsum(
        "bhij,bjhd->bihd", p, V, preferred_element_type=jnp.float32
    )
    o = o.astype(jnp.bfloat16).reshape(B, Sq, Hq * DH)
    partial = jnp.dot(
        o, Wo.astype(jnp.bfloat16), preferred_element_type=jnp.float32
    )

    out = _butterfly_allreduce(partial.astype(jnp.bfloat16).reshape(B * Sq, D))
    return out.reshape(B, Sq, D)
